# Initial kernel scaffold; baseline (speedup 1.0000x reference)
import functools

import jax
import jax.numpy as jnp
from jax import lax
from jax.experimental import pallas as pl
from jax.experimental.pallas import tpu as pltpu
from jax.experimental.pallas import tpu_sc as plsc


def _lse_body(table_ref, lse_ref):
    t = table_ref[...]
    m = jnp.max(t, axis=1, keepdims=True)
    lse_ref[...] = (m + jnp.log(jnp.sum(jnp.exp(t - m), axis=1, keepdims=True)))[:, 0]


def _compute_lse(table):
    v = table.shape[0]
    return pl.pallas_call(
        _lse_body,
        out_shape=jax.ShapeDtypeStruct((v,), jnp.float32),
    )(table)


_INFO = plsc.get_sparse_core_info()
_NC, _NS, _L = _INFO.num_cores, _INFO.num_subcores, _INFO.num_lanes
_NW = _NC * _NS


@functools.lru_cache(maxsize=None)
def _make_sc_gather(n, vocab, d, c):
    assert n % (_NW * c) == 0 and c % _L == 0
    b_per_w = n // _NW
    nch = b_per_w // c
    mesh = plsc.VectorSubcoreMesh(core_axis_name="c", subcore_axis_name="s")

    @functools.partial(
        pl.kernel,
        mesh=mesh,
        out_type=(
            jax.ShapeDtypeStruct((n, d), jnp.float32),
            jax.ShapeDtypeStruct((_NW, _L), jnp.float32),
        ),
        scratch_types=[
            pltpu.VMEM((c,), jnp.int32),
            pltpu.VMEM((c,), jnp.int32),
            pltpu.VMEM((c, d), jnp.float32),
            pltpu.VMEM((vocab,), jnp.float32),
            pltpu.VMEM((_L,), jnp.float32),
            pltpu.SemaphoreType.DMA,
        ],
    )
    def sc_kernel(idx_hbm, tgt_hbm, table_hbm, lse_hbm, out_hbm, part_hbm,
                  idx_v, tgt_v, rows_v, lse_v, acc_v, sem):
        wid = lax.axis_index("s") * _NC + lax.axis_index("c")
        w_base = wid * b_per_w
        pltpu.sync_copy(lse_hbm, lse_v)

        def body(g, acc):
            base = w_base + g * c
            pltpu.sync_copy(idx_hbm.at[pl.ds(base, c)], idx_v)
            pltpu.sync_copy(tgt_hbm.at[pl.ds(base, c)], tgt_v)
            pltpu.async_copy(table_hbm.at[idx_v], rows_v, sem).wait()
            pltpu.sync_copy(rows_v, out_hbm.at[pl.ds(base, c)])
            for j in range(c // _L):
                iv = idx_v[pl.ds(j * _L, _L)]
                tv = tgt_v[pl.ds(j * _L, _L)]
                rid = lax.iota(jnp.int32, _L) + j * _L
                lse_g = plsc.load_gather(lse_v, [iv])
                val_g = plsc.load_gather(rows_v, [rid, tv])
                acc = acc + (lse_g - val_g)
            return acc

        acc = lax.fori_loop(0, nch, body, jnp.zeros((_L,), jnp.float32))
        acc_v[...] = acc
        pltpu.sync_copy(acc_v, part_hbm.at[wid])

    return sc_kernel


def kernel(idx, targets, table):
    b, t = idx.shape
    vocab, d = table.shape
    n = b * t
    idx_f = idx.reshape(n).astype(jnp.int32)
    tgt_f = targets.reshape(n).astype(jnp.int32)
    lse = _compute_lse(table)
    out_flat, part = _make_sc_gather(n, vocab, d, 64)(idx_f, tgt_f, table, lse)
    logits = out_flat.reshape(b, t, d)
    loss = jnp.sum(part) / n
    return (logits, loss)


# speedup vs baseline: 1.3700x; 1.3700x over previous
import functools

import jax
import jax.numpy as jnp
from jax import lax
from jax.experimental import pallas as pl
from jax.experimental.pallas import tpu as pltpu
from jax.experimental.pallas import tpu_sc as plsc


def _lse_body(table_ref, lse_ref):
    t = table_ref[...]
    m = jnp.max(t, axis=1, keepdims=True)
    lse_ref[...] = (m + jnp.log(jnp.sum(jnp.exp(t - m), axis=1, keepdims=True)))[:, 0]


def _compute_lse(table):
    v = table.shape[0]
    return pl.pallas_call(
        _lse_body,
        out_shape=jax.ShapeDtypeStruct((v,), jnp.float32),
    )(table)


_INFO = plsc.get_sparse_core_info()
_NC, _NS, _L = _INFO.num_cores, _INFO.num_subcores, _INFO.num_lanes
_NW = _NC * _NS


@functools.lru_cache(maxsize=None)
def _make_sc_gather(n, vocab, d, c):
    assert n % (_NW * c) == 0 and c % _L == 0
    b_per_w = n // _NW
    nch = b_per_w // c
    mesh = plsc.VectorSubcoreMesh(core_axis_name="c", subcore_axis_name="s")

    @functools.partial(
        pl.kernel,
        mesh=mesh,
        compiler_params=pltpu.CompilerParams(
            needs_layout_passes=False, use_tc_tiling_on_sc=False
        ),
        out_type=(
            jax.ShapeDtypeStruct((n, d), jnp.float32),
            jax.ShapeDtypeStruct((_NW, _L), jnp.float32),
        ),
        scratch_types=[
            pltpu.VMEM((c,), jnp.int32),
            pltpu.VMEM((c,), jnp.int32),
            pltpu.VMEM((c, d), jnp.float32),
            pltpu.VMEM((vocab,), jnp.float32),
            pltpu.VMEM((_L,), jnp.float32),
            pltpu.SemaphoreType.DMA,
        ],
    )
    def sc_kernel(idx_hbm, tgt_hbm, table_hbm, lse_hbm, out_hbm, part_hbm,
                  idx_v, tgt_v, rows_v, lse_v, acc_v, sem):
        wid = lax.axis_index("s") * _NC + lax.axis_index("c")
        w_base = wid * b_per_w
        pltpu.sync_copy(lse_hbm, lse_v)

        def body(g, acc):
            base = w_base + g * c
            pltpu.sync_copy(idx_hbm.at[pl.ds(base, c)], idx_v)
            pltpu.sync_copy(tgt_hbm.at[pl.ds(base, c)], tgt_v)
            pltpu.async_copy(table_hbm.at[idx_v], rows_v, sem).wait()
            pltpu.sync_copy(rows_v, out_hbm.at[pl.ds(base, c)])
            for j in range(c // _L):
                iv = idx_v[pl.ds(j * _L, _L)]
                tv = tgt_v[pl.ds(j * _L, _L)]
                rid = lax.iota(jnp.int32, _L) + j * _L
                lse_g = plsc.load_gather(lse_v, [iv])
                val_g = plsc.load_gather(rows_v, [rid, tv])
                acc = acc + (lse_g - val_g)
            return acc

        acc = lax.fori_loop(0, nch, body, jnp.zeros((_L,), jnp.float32))
        acc_v[...] = acc
        pltpu.sync_copy(acc_v, part_hbm.at[wid])

    return sc_kernel


def kernel(idx, targets, table):
    b, t = idx.shape
    vocab, d = table.shape
    n = b * t
    idx_f = idx.reshape(n).astype(jnp.int32)
    tgt_f = targets.reshape(n).astype(jnp.int32)
    lse = _compute_lse(table)
    out_flat, part = _make_sc_gather(n, vocab, d, 64)(idx_f, tgt_f, table, lse)
    logits = out_flat.reshape(b, t, d)
    loss = jnp.sum(part) / n
    return (logits, loss)


# speedup vs baseline: 1.4327x; 1.0457x over previous
import functools

import jax
import jax.numpy as jnp
from jax import lax
from jax.experimental import pallas as pl
from jax.experimental.pallas import tpu as pltpu
from jax.experimental.pallas import tpu_sc as plsc


def _lse_body(table_ref, lse_ref):
    t = table_ref[...]
    m = jnp.max(t, axis=1, keepdims=True)
    lse_ref[...] = (m + jnp.log(jnp.sum(jnp.exp(t - m), axis=1, keepdims=True)))[:, 0]


def _compute_lse(table):
    v = table.shape[0]
    return pl.pallas_call(
        _lse_body,
        out_shape=jax.ShapeDtypeStruct((v,), jnp.float32),
    )(table)


_INFO = plsc.get_sparse_core_info()
_NC, _NS, _L = _INFO.num_cores, _INFO.num_subcores, _INFO.num_lanes
_NW = _NC * _NS


@functools.lru_cache(maxsize=None)
def _make_sc_gather(n, vocab, d, c):
    assert c % _L == 0
    b_per_w = n // _NW
    nch = b_per_w // c
    assert n == _NW * nch * c and nch % 2 == 0 and nch >= 4
    mesh = plsc.VectorSubcoreMesh(core_axis_name="c", subcore_axis_name="s")

    @functools.partial(
        pl.kernel,
        mesh=mesh,
        compiler_params=pltpu.CompilerParams(
            needs_layout_passes=False, use_tc_tiling_on_sc=False
        ),
        out_type=(
            jax.ShapeDtypeStruct((n, d), jnp.float32),
            jax.ShapeDtypeStruct((_NW, _L), jnp.float32),
        ),
        scratch_types=[
            pltpu.VMEM((b_per_w,), jnp.int32),
            pltpu.VMEM((b_per_w,), jnp.int32),
            pltpu.VMEM((c, d), jnp.float32),
            pltpu.VMEM((c, d), jnp.float32),
            pltpu.VMEM((vocab,), jnp.float32),
            pltpu.VMEM((_L,), jnp.float32),
            pltpu.SemaphoreType.DMA,
            pltpu.SemaphoreType.DMA,
            pltpu.SemaphoreType.DMA,
            pltpu.SemaphoreType.DMA,
        ],
    )
    def sc_kernel(idx_hbm, tgt_hbm, table_hbm, lse_hbm, out_hbm, part_hbm,
                  idx_v, tgt_v, rows0, rows1, lse_v, acc_v, g0, g1, w0, w1):
        wid = lax.axis_index("s") * _NC + lax.axis_index("c")
        w_base = wid * b_per_w
        rows = (rows0, rows1)
        gsem = (g0, g1)
        wsem = (w0, w1)
        pltpu.sync_copy(lse_hbm, lse_v)
        pltpu.sync_copy(idx_hbm.at[pl.ds(w_base, b_per_w)], idx_v)
        pltpu.sync_copy(tgt_hbm.at[pl.ds(w_base, b_per_w)], tgt_v)

        def start_gather(g, b):
            pltpu.async_copy(
                table_hbm.at[idx_v.at[pl.ds(g * c, c)]], rows[b], gsem[b])

        def wait_gather(b):
            pltpu.make_async_copy(
                table_hbm.at[pl.ds(0, c)], rows[b], gsem[b]).wait()

        def start_write(g, b):
            pltpu.async_copy(
                rows[b], out_hbm.at[pl.ds(w_base + g * c, c)], wsem[b])

        def wait_write(b):
            pltpu.make_async_copy(
                rows[b], out_hbm.at[pl.ds(0, c)], wsem[b]).wait()

        def loss_chunk(g, b, acc):
            for j in range(c // _L):
                iv = idx_v[pl.ds(g * c + j * _L, _L)]
                tv = tgt_v[pl.ds(g * c + j * _L, _L)]
                rid = lax.iota(jnp.int32, _L) + j * _L
                lse_g = plsc.load_gather(lse_v, [iv])
                val_g = plsc.load_gather(rows[b], [rid, tv])
                acc = acc + (lse_g - val_g)
            return acc

        start_gather(0, 0)
        start_gather(1, 1)

        def body(gg, acc):
            for b in range(2):
                g = gg * 2 + b
                wait_gather(b)
                start_write(g, b)
                acc = loss_chunk(g, b, acc)
                wait_write(b)
                start_gather(g + 2, b)
            return acc

        acc = lax.fori_loop(0, nch // 2 - 1, body,
                            jnp.zeros((_L,), jnp.float32))
        for b in range(2):
            g = nch - 2 + b
            wait_gather(b)
            start_write(g, b)
            acc = loss_chunk(g, b, acc)
            wait_write(b)

        acc_v[...] = acc
        pltpu.sync_copy(acc_v, part_hbm.at[wid])

    return sc_kernel


def kernel(idx, targets, table):
    b, t = idx.shape
    vocab, d = table.shape
    n = b * t
    idx_f = idx.reshape(n).astype(jnp.int32)
    tgt_f = targets.reshape(n).astype(jnp.int32)
    lse = _compute_lse(table)
    out_flat, part = _make_sc_gather(n, vocab, d, 32)(idx_f, tgt_f, table, lse)
    logits = out_flat.reshape(b, t, d)
    loss = jnp.sum(part) / n
    return (logits, loss)


# speedup vs baseline: 2.6985x; 1.8835x over previous
import functools

import jax
import jax.numpy as jnp
from jax import lax
from jax.experimental import pallas as pl
from jax.experimental.pallas import tpu as pltpu
from jax.experimental.pallas import tpu_sc as plsc


def _lse_body(table_ref, lse_ref):
    t = table_ref[...]
    m = jnp.max(t, axis=1, keepdims=True)
    lse = (m + jnp.log(jnp.sum(jnp.exp(t - m), axis=1, keepdims=True)))[:, 0]
    lse_ref[...] = jnp.pad(lse, (0, lse_ref.shape[0] - lse.shape[0]))


def _compute_lse(table, vp):
    return pl.pallas_call(
        _lse_body,
        out_shape=jax.ShapeDtypeStruct((vp,), jnp.float32),
    )(table)


_INFO = plsc.get_sparse_core_info()
_NC, _NS, _L = _INFO.num_cores, _INFO.num_subcores, _INFO.num_lanes
_NW = _NC * _NS


@functools.lru_cache(maxsize=None)
def _make_sc_kernel(bsz, t, v, d):
    sl = (d + 127) // 128
    dp = sl * 128
    n = bsz * t
    nb = bsz // _NW
    tokw = nb * t
    tp = ((t + _L - 1) // _L) * _L
    ngrp = tokw // _L
    assert bsz == nb * _NW and nb % 2 == 0 and tokw % _L == 0
    mesh = plsc.VectorSubcoreMesh(core_axis_name="c", subcore_axis_name="s")

    @functools.partial(
        pl.kernel,
        mesh=mesh,
        compiler_params=pltpu.CompilerParams(
            needs_layout_passes=False, use_tc_tiling_on_sc=True
        ),
        out_type=(
            jax.ShapeDtypeStruct((bsz, t, d), jnp.float32),
            jax.ShapeDtypeStruct((bsz, t, 128), jnp.float32),
            jax.ShapeDtypeStruct((_NW * _L,), jnp.float32),
        ),
        scratch_types=[
            pltpu.VMEM((tokw + _L * 4,), jnp.int32),
            pltpu.VMEM((tokw,), jnp.int32),
            pltpu.VMEM((nb * tp,), jnp.int32),
            pltpu.VMEM((t, sl, 128), jnp.float32),
            pltpu.VMEM((t, sl, 128), jnp.float32),
            pltpu.VMEM((dp,), jnp.float32),
            pltpu.VMEM((tokw,), jnp.int32),
            pltpu.VMEM((tokw,), jnp.float32),
            pltpu.VMEM((_L,), jnp.float32),
            pltpu.SemaphoreType.DMA,
            pltpu.SemaphoreType.DMA,
            pltpu.SemaphoreType.DMA,
            pltpu.SemaphoreType.DMA,
            pltpu.SemaphoreType.DMA,
        ],
    )
    def sc_kernel(idx_hbm, tgt_hbm, tab3_hbm, tabflat_hbm, lse_hbm,
                  out_hbm, tail_hbm, part_hbm,
                  idxw, tgtw, idx_p, rows0, rows1,
                  lse_v, fi_v, vals_v,
                  acc_v, g0, g1, w0, w1, vsem):
        wid = lax.axis_index("s") * _NC + lax.axis_index("c")
        w_tok0 = wid * tokw
        w_b0 = wid * nb
        rows = (rows0, rows1)
        gsem = (g0, g1)
        wsem = (w0, w1)

        pltpu.sync_copy(lse_hbm, lse_v)
        pltpu.sync_copy(idx_hbm.at[pl.ds(w_tok0, tokw)],
                        idxw.at[pl.ds(0, tokw)])
        pltpu.sync_copy(tgt_hbm.at[pl.ds(w_tok0, tokw)], tgtw)
        for j in range(4):
            idxw[pl.ds(tokw + j * _L, _L)] = jnp.zeros((_L,), jnp.int32)

        def stage_body(q, carry):
            for j in range(tp // _L):
                idx_p[pl.ds(q * tp + j * _L, _L)] = (
                    idxw[pl.ds(q * t + j * _L, _L)])
            return carry

        lax.fori_loop(0, nb, stage_body, 0)

        def fi_body(g, carry):
            iv = idxw[pl.ds(g * _L, _L)]
            tv = tgtw[pl.ds(g * _L, _L)]
            fi_v[pl.ds(g * _L, _L)] = iv * dp + tv
            return carry

        lax.fori_loop(0, ngrp, fi_body, 0)
        pltpu.async_copy(tabflat_hbm.at[fi_v], vals_v, vsem)

        def start_gather(q, b):
            pltpu.async_copy(
                tab3_hbm.at[idx_p.at[pl.ds(q * tp, t)]], rows[b], gsem[b])

        def wait_gather(b):
            pltpu.make_async_copy(
                tab3_hbm.at[pl.ds(0, t)], rows[b], gsem[b]).wait()

        def writes(q, b, start):
            bg = w_b0 + q
            for cb in range(sl):
                src = rows[b].at[:, cb, pl.ds(0, 128)]
                if cb * 128 + 128 <= d:
                    dst = out_hbm.at[bg, :, pl.ds(cb * 128, 128)]
                else:
                    dst = tail_hbm.at[bg]
                if start:
                    pltpu.async_copy(src, dst, wsem[b])
                else:
                    pltpu.make_async_copy(src, dst, wsem[b]).wait()

        start_gather(0, 0)
        start_gather(1, 1)

        def body(gg, carry):
            for b in range(2):
                q = gg * 2 + b
                wait_gather(b)
                writes(q, b, True)
                writes(q, b, False)
                start_gather(q + 2, b)
            return carry

        lax.fori_loop(0, nb // 2 - 1, body, 0)
        for b in range(2):
            q = nb - 2 + b
            wait_gather(b)
            writes(q, b, True)
            writes(q, b, False)

        pltpu.make_async_copy(
            tabflat_hbm.at[pl.ds(0, tokw)], vals_v, vsem).wait()

        def loss_body(g, acc):
            iv = idxw[pl.ds(g * _L, _L)]
            lse_g = plsc.load_gather(lse_v, [iv])
            vv = vals_v[pl.ds(g * _L, _L)]
            return acc + (lse_g - vv)

        acc = lax.fori_loop(0, ngrp, loss_body, jnp.zeros((_L,), jnp.float32))
        acc_v[...] = acc
        pltpu.sync_copy(acc_v, part_hbm.at[pl.ds(wid * _L, _L)])

    return sc_kernel


def kernel(idx, targets, table):
    bsz, t = idx.shape
    v, d = table.shape
    sl = (d + 127) // 128
    dp = sl * 128
    n = bsz * t
    idx_f = idx.reshape(n).astype(jnp.int32)
    tgt_f = targets.reshape(n).astype(jnp.int32)
    tab3 = jnp.pad(table, ((0, 0), (0, dp - d))).reshape(v, sl, 128)
    tabflat = tab3.reshape(v * dp)
    lse = _compute_lse(table, dp)
    main, tail, part = _make_sc_kernel(bsz, t, v, d)(
        idx_f, tgt_f, tab3, tabflat, lse)
    ntile = (d // 128) * 128
    logits = lax.dynamic_update_slice(
        main, tail[:, :, : d - ntile], (0, 0, ntile))
    loss = jnp.sum(part) / n
    return (logits, loss)
